# baseline (speedup 1.0000x reference)
import functools

import jax
import jax.numpy as jnp
import numpy as np
from jax import lax
from jax.experimental import pallas as pl
from jax.experimental.pallas import tpu as pltpu
from jax.experimental.pallas import tpu_sc as plsc

_FIELD_DIMS = [100000] * 26
_EMBED_DIM = 64
_OFFS = np.array((0, *np.cumsum(_FIELD_DIMS)[:-1]), dtype=np.int32)

_NC = 2
_NS = 16
_NW = _NC * _NS
_LANES = 16
_CHUNK = 128


@functools.lru_cache(maxsize=None)
def _sc_gather(B, D, nchunk):
    mesh = plsc.VectorSubcoreMesh(core_axis_name="c", subcore_axis_name="s")

    @functools.partial(
        pl.kernel,
        mesh=mesh,
        out_type=jax.ShapeDtypeStruct((B, D), jnp.float32),
        scratch_types=[
            pltpu.VMEM((nchunk, _CHUNK), jnp.int32),
            pltpu.VMEM((nchunk, _CHUNK), jnp.int32),
            pltpu.VMEM((nchunk, _CHUNK), jnp.int32),
            pltpu.VMEM((_CHUNK, D), jnp.float32),
            pltpu.VMEM((_CHUNK, D), jnp.float32),
            pltpu.SemaphoreType.DMA,
            pltpu.SemaphoreType.DMA,
        ],
        compiler_params=pltpu.CompilerParams(use_tc_tiling_on_sc=False),
    )
    def k(x_hbm, off_hbm, table_hbm, out_hbm, x_v, off_v, idx_v, r0, r1, g0, g1):
        wid = lax.axis_index("s") * _NC + lax.axis_index("c")
        base = wid * (nchunk * _CHUNK)

        pltpu.sync_copy(x_hbm.at[wid], x_v)
        pltpu.sync_copy(off_hbm, off_v)

        def add_body(c, carry):
            for j in range(_CHUNK // _LANES):
                sl = pl.ds(j * _LANES, _LANES)
                idx_v[c, sl] = x_v[c, sl] + off_v[c, sl]
            return carry

        lax.fori_loop(0, nchunk, add_body, 0)

        def chunk_body(c, carry):
            pltpu.async_copy(table_hbm.at[idx_v.at[c]], r0, g0).wait()
            pltpu.sync_copy(r0, out_hbm.at[pl.ds(base + c * _CHUNK, _CHUNK)])
            return carry

        lax.fori_loop(0, nchunk, chunk_body, 0)

    return k


def kernel(x, table):
    batch, nf = x.shape
    D = table.shape[1]
    B = batch * nf
    bpw = B // _NW
    nchunk = bpw // _CHUNK
    assert bpw % _CHUNK == 0 and bpw % nf == 0

    off_flat = np.tile(_OFFS, bpw // nf).reshape(nchunk, _CHUNK)
    x2 = x.reshape(_NW, nchunk, _CHUNK)
    out = _sc_gather(B, D, nchunk)(x2, jnp.asarray(off_flat), table)
    return out.reshape(batch, nf, D)
